# baseline (speedup 1.0000x reference)
import functools

import jax
import jax.numpy as jnp
from jax import lax
from jax.experimental import pallas as pl
from jax.experimental.pallas import tpu as pltpu
from jax.experimental.pallas import tpu_sc as plsc

DIM = 32
BATCH = 16384
NC = 2
NS = 16
L = 16
NW = NC * NS
BPW = BATCH // NW
CHUNKS = BPW // L


def _bpr_body(uid_hbm, iid_hbm, uembT_hbm, iembT_hbm, ub_hbm, ib_hbm, out_hbm,
              uid_v, iid_v, ud_v, id_v, ubv, ibv, dotv,
              sem_u, sem_i, sem_ub, sem_ib):
    wid = lax.axis_index("s") * NC + lax.axis_index("c")
    base = wid * BPW

    pltpu.sync_copy(uid_hbm.at[pl.ds(base, BPW)], uid_v)
    pltpu.sync_copy(iid_hbm.at[pl.ds(base, BPW)], iid_v)

    cub = pltpu.async_copy(ub_hbm.at[uid_v], ubv, sem_ub)
    cib = pltpu.async_copy(ib_hbm.at[iid_v], ibv, sem_ib)

    def fire(d, carry):
        pltpu.async_copy(uembT_hbm.at[d].at[uid_v], ud_v.at[d], sem_u)
        pltpu.async_copy(iembT_hbm.at[d].at[iid_v], id_v.at[d], sem_i)
        return carry

    lax.fori_loop(0, DIM, fire, 0)

    pltpu.make_async_copy(uembT_hbm.at[:, pl.ds(0, BPW)], ud_v, sem_u).wait()
    pltpu.make_async_copy(iembT_hbm.at[:, pl.ds(0, BPW)], id_v, sem_i).wait()
    cub.wait()
    cib.wait()

    def chunk(c, carry):
        o = c * L
        acc = ubv[pl.ds(o, L)] + ibv[pl.ds(o, L)]
        for d in range(DIM):
            acc = acc + ud_v[d, pl.ds(o, L)] * id_v[d, pl.ds(o, L)]
        dotv[pl.ds(o, L)] = acc
        return carry

    lax.fori_loop(0, CHUNKS, chunk, 0)
    pltpu.sync_copy(dotv, out_hbm.at[pl.ds(base, BPW)])


@jax.jit
def kernel(user_ids, item_ids, user_emb, item_emb, user_bias, item_bias):
    uid = user_ids.astype(jnp.int32)
    iid = item_ids.astype(jnp.int32)
    mesh = plsc.VectorSubcoreMesh(core_axis_name="c", subcore_axis_name="s")
    run = functools.partial(
        pl.kernel,
        mesh=mesh,
        compiler_params=pltpu.CompilerParams(
            needs_layout_passes=False, use_tc_tiling_on_sc=False),
        out_type=jax.ShapeDtypeStruct((BATCH,), jnp.float32),
        scratch_types=[
            pltpu.VMEM((BPW,), jnp.int32),
            pltpu.VMEM((BPW,), jnp.int32),
            pltpu.VMEM((DIM, BPW), jnp.float32),
            pltpu.VMEM((DIM, BPW), jnp.float32),
            pltpu.VMEM((BPW,), jnp.float32),
            pltpu.VMEM((BPW,), jnp.float32),
            pltpu.VMEM((BPW,), jnp.float32),
            pltpu.SemaphoreType.DMA,
            pltpu.SemaphoreType.DMA,
            pltpu.SemaphoreType.DMA,
            pltpu.SemaphoreType.DMA,
        ],
    )(_bpr_body)
    return run(uid, iid, user_emb.T, item_emb.T,
               user_bias.reshape(-1), item_bias.reshape(-1))


# speedup vs baseline: 5.6979x; 5.6979x over previous
import functools

import jax
import jax.numpy as jnp
from jax import lax
from jax.experimental import pallas as pl
from jax.experimental.pallas import tpu as pltpu
from jax.experimental.pallas import tpu_sc as plsc

DIM = 32
BATCH = 16384
NC = 2
NS = 16
L = 16
NW = NC * NS
BPW = BATCH // NW
Q = 128
NQ = BPW // Q
PACK = 128 // DIM


def _bpr_body(uid_hbm, iid_hbm, upack_hbm, ipack_hbm, ub_hbm, ib_hbm, out_hbm,
              uid_v, iid_v, uk_v, ik_v, urow0, urow1, irow0, irow1,
              ubv, ibv, dotv, sem_u0, sem_u1, sem_i0, sem_i1, sem_ub, sem_ib):
    wid = lax.axis_index("s") * NC + lax.axis_index("c")
    base = wid * BPW

    pltpu.sync_copy(uid_hbm.at[pl.ds(base, BPW)], uid_v)
    pltpu.sync_copy(iid_hbm.at[pl.ds(base, BPW)], iid_v)

    cub = pltpu.async_copy(ub_hbm.at[uid_v], ubv, sem_ub)
    cib = pltpu.async_copy(ib_hbm.at[iid_v], ibv, sem_ib)

    def mkidx(c, carry):
        o = c * L
        uk_v[pl.ds(o, L)] = uid_v[pl.ds(o, L)] >> 2
        ik_v[pl.ds(o, L)] = iid_v[pl.ds(o, L)] >> 2
        return carry

    lax.fori_loop(0, BPW // L, mkidx, 0)

    ubufs = [urow0, urow1]
    ibufs = [irow0, irow1]
    usems = [sem_u0, sem_u1]
    isems = [sem_i0, sem_i1]

    def fire(q):
        b = q % 2
        cu = pltpu.async_copy(
            upack_hbm.at[uk_v.at[pl.ds(q * Q, Q)]], ubufs[b], usems[b])
        ci = pltpu.async_copy(
            ipack_hbm.at[ik_v.at[pl.ds(q * Q, Q)]], ibufs[b], isems[b])
        return cu, ci

    iota = lax.iota(jnp.int32, L)

    def extract(q):
        b = q % 2
        ub, ib = ubufs[b], ibufs[b]
        for g in range(Q // L):
            o = q * Q + g * L
            u16 = uid_v[pl.ds(o, L)]
            i16 = iid_v[pl.ds(o, L)]
            row = g * L + iota
            ucol0 = (u16 & (PACK - 1)) * DIM
            icol0 = (i16 & (PACK - 1)) * DIM
            acc = ubv[pl.ds(o, L)] + ibv[pl.ds(o, L)]
            for d in range(DIM):
                acc = acc + (plsc.load_gather(ub, [row, ucol0 + d])
                             * plsc.load_gather(ib, [row, icol0 + d]))
            dotv[pl.ds(o, L)] = acc

    pend = [fire(0), fire(1)]
    cub.wait()
    cib.wait()
    for q in range(NQ):
        cu, ci = pend[q % 2]
        cu.wait()
        ci.wait()
        extract(q)
        if q + 2 < NQ:
            pend[q % 2] = fire(q + 2)

    pltpu.sync_copy(dotv, out_hbm.at[pl.ds(base, BPW)])


@jax.jit
def kernel(user_ids, item_ids, user_emb, item_emb, user_bias, item_bias):
    uid = user_ids.astype(jnp.int32)
    iid = item_ids.astype(jnp.int32)
    mesh = plsc.VectorSubcoreMesh(core_axis_name="c", subcore_axis_name="s")
    run = functools.partial(
        pl.kernel,
        mesh=mesh,
        compiler_params=pltpu.CompilerParams(
            needs_layout_passes=False, use_tc_tiling_on_sc=False),
        out_type=jax.ShapeDtypeStruct((BATCH,), jnp.float32),
        scratch_types=[
            pltpu.VMEM((BPW,), jnp.int32),
            pltpu.VMEM((BPW,), jnp.int32),
            pltpu.VMEM((BPW,), jnp.int32),
            pltpu.VMEM((BPW,), jnp.int32),
            pltpu.VMEM((Q, 128), jnp.float32),
            pltpu.VMEM((Q, 128), jnp.float32),
            pltpu.VMEM((Q, 128), jnp.float32),
            pltpu.VMEM((Q, 128), jnp.float32),
            pltpu.VMEM((BPW,), jnp.float32),
            pltpu.VMEM((BPW,), jnp.float32),
            pltpu.VMEM((BPW,), jnp.float32),
            pltpu.SemaphoreType.DMA,
            pltpu.SemaphoreType.DMA,
            pltpu.SemaphoreType.DMA,
            pltpu.SemaphoreType.DMA,
            pltpu.SemaphoreType.DMA,
            pltpu.SemaphoreType.DMA,
        ],
    )(_bpr_body)
    return run(uid, iid,
               user_emb.reshape(250000, 128), item_emb.reshape(250000, 128),
               user_bias.reshape(-1), item_bias.reshape(-1))
